# baseline (speedup 1.0000x reference)
import functools

import jax
import jax.numpy as jnp
from jax import lax
from jax.experimental import pallas as pl
from jax.experimental.pallas import tpu as pltpu
from jax.experimental.pallas import tpu_sc as plsc

NUM_NUMERIC = 64
EMB = 64
BATCH = 4096
FEAT = 128
ROWS = 100000

_NC = 2
_NS = 16
_NW = _NC * _NS
_BPW = BATCH // _NW

_TBLK = 16384
_NBLK = (ROWS + _TBLK - 1) // _TBLK
_PROWS = _NBLK * (_TBLK // 4)
_PW = 4 * EMB
_BSH = _TBLK.bit_length() - 1
_QSH = _BSH - 2
_QMASK = (_TBLK // 4) - 1


def _retile_body(eye, tabT, out):
  q = _TBLK // 4
  a = tabT[...]
  packed = jnp.concatenate([a[:, k * q:(k + 1) * q] for k in range(4)],
                           axis=0)
  out[...] = lax.dot_general(
      packed, eye[...], dimension_numbers=(((0,), (0,)), ((), ())),
      preferred_element_type=jnp.float32)


@jax.jit
def _retile(tabT):
  eye = jnp.eye(_PW, dtype=jnp.float32)
  return pl.pallas_call(
      _retile_body,
      grid=(_NBLK,),
      in_specs=[pl.BlockSpec((_PW, _PW), lambda i: (0, 0)),
                pl.BlockSpec((EMB, _TBLK), lambda i: (0, i))],
      out_specs=pl.BlockSpec((_TBLK // 4, _PW), lambda i: (i, 0)),
      out_shape=jax.ShapeDtypeStruct((_PROWS, _PW), jnp.float32),
  )(eye, tabT)


def _sc_gather_body(id_hbm, tab_hbm, out_hbm, idx_v, rows, sem):
  wid = lax.axis_index("s") * _NC + lax.axis_index("c")
  base = wid * _BPW
  pltpu.sync_copy(id_hbm.at[pl.ds(base, _BPW)], idx_v)
  for c in range(_BPW // 16):
    s = pl.ds(c * 16, 16)
    v = idx_v[s]
    idx_v[s] = jnp.bitwise_or(
        lax.shift_left(lax.shift_right_logical(v, _BSH), _QSH),
        jnp.bitwise_and(v, _QMASK))
  pltpu.async_copy(tab_hbm.at[idx_v], rows, sem).wait()
  pltpu.sync_copy(rows, out_hbm.at[pl.ds(base, _BPW)])


@jax.jit
def _sc_gather(idx, tab):
  mesh = plsc.VectorSubcoreMesh(core_axis_name="c", subcore_axis_name="s")
  return pl.kernel(
      _sc_gather_body,
      mesh=mesh,
      compiler_params=pltpu.CompilerParams(needs_layout_passes=False),
      out_type=jax.ShapeDtypeStruct((BATCH, _PW), jnp.float32),
      scratch_types=[
          pltpu.VMEM((_BPW,), jnp.int32),
          pltpu.VMEM((_BPW, _PW), jnp.float32),
          pltpu.SemaphoreType.DMA,
      ],
  )(idx, tab)


def _quad_select(qid, pack):
  q = jnp.bitwise_and(lax.shift_right_logical(qid[:], _QSH), 3)[:, None]
  v01 = jnp.where(q == 0, pack[:, 0:EMB], pack[:, EMB:2 * EMB])
  v23 = jnp.where(q == 2, pack[:, 2 * EMB:3 * EMB], pack[:, 3 * EMB:4 * EMB])
  return jnp.where(q < 2, v01, v23)


def _mlp_body(uid, pid, upack, ppack, ff, wnum, bnum, wsty, bsty, w1, b1, w2,
              b2, w3, b3, w4, b4, out):
  f32 = jnp.float32
  uvec = _quad_select(uid, upack)
  pvec = _quad_select(pid, ppack)
  numeric = jnp.maximum(
      jnp.dot(ff[:, :NUM_NUMERIC], wnum[:], preferred_element_type=f32)
      + bnum[:], 0.0)
  style = jnp.maximum(
      jnp.dot(ff[:, NUM_NUMERIC:], wsty[:], preferred_element_type=f32)
      + bsty[:], 0.0)
  h = (jnp.dot(uvec, w1[0:EMB], preferred_element_type=f32)
       + jnp.dot(pvec, w1[EMB:2 * EMB], preferred_element_type=f32)
       + jnp.dot(numeric, w1[2 * EMB:3 * EMB], preferred_element_type=f32)
       + jnp.dot(style, w1[3 * EMB:4 * EMB], preferred_element_type=f32)
       + b1[:])
  h = jnp.maximum(h, 0.0)
  h = jnp.maximum(jnp.dot(h, w2[:], preferred_element_type=f32) + b2[:], 0.0)
  h = jnp.maximum(jnp.dot(h, w3[:], preferred_element_type=f32) + b3[:], 0.0)
  logit = jnp.sum(h * w4[:], axis=1, keepdims=True) + b4[:]
  out[:] = 1.0 / (1.0 + jnp.exp(-logit))


@jax.jit
def _mlp(uid, pid, upack, ppack, ff, wnum, bnum, wsty, bsty, w1, b1, w2, b2,
         w3, b3, w4, b4):
  nblk = 4
  blk = BATCH // nblk
  row_spec = lambda width: pl.BlockSpec((blk, width), lambda i: (i, 0))
  id_spec = pl.BlockSpec((blk,), lambda i: (i,))
  full = lambda a: pl.BlockSpec(a.shape, lambda i: tuple(0 for _ in a.shape))
  return pl.pallas_call(
      _mlp_body,
      grid=(nblk,),
      in_specs=[
          id_spec,
          id_spec,
          row_spec(_PW),
          row_spec(_PW),
          row_spec(FEAT),
          full(wnum), full(bnum), full(wsty), full(bsty),
          full(w1), full(b1), full(w2), full(b2), full(w3), full(b3),
          full(w4), full(b4),
      ],
      out_specs=pl.BlockSpec((blk, 1), lambda i: (i, 0)),
      out_shape=jax.ShapeDtypeStruct((BATCH, 1), jnp.float32),
  )(uid, pid, upack, ppack, ff, wnum, bnum, wsty, bsty, w1, b1, w2, b2, w3,
    b3, w4, b4)


def kernel(user_id, product_id, full_features, user_table, product_table,
           W_num, b_num, W_style, b_style, W1, b1, W2, b2, W3, b3, W4, b4):
  uid = user_id.astype(jnp.int32)
  pid = product_id.astype(jnp.int32)
  utabT = jnp.swapaxes(user_table, 0, 1)
  ptabT = jnp.swapaxes(product_table, 0, 1)
  utab = _retile(utabT)
  upack = _sc_gather(uid, utab)
  ptab = _retile(ptabT)
  ppack = _sc_gather(pid, ptab)
  return _mlp(uid, pid, upack, ppack, full_features,
              W_num, b_num.reshape(1, EMB), W_style, b_style.reshape(1, EMB),
              W1, b1.reshape(1, 128), W2, b2.reshape(1, 64),
              W3, b3.reshape(1, 32), W4.reshape(1, 32), b4.reshape(1, 1))


# speedup vs baseline: 1.0395x; 1.0395x over previous
import functools

import jax
import jax.numpy as jnp
from jax import lax
from jax.experimental import pallas as pl
from jax.experimental.pallas import tpu as pltpu
from jax.experimental.pallas import tpu_sc as plsc

NUM_NUMERIC = 64
EMB = 64
BATCH = 4096
FEAT = 128
ROWS = 100000

_NC = 2
_NS = 16
_NW = _NC * _NS
_BPW = BATCH // _NW

_TBLK = 8192
_NBLK = (ROWS + _TBLK - 1) // _TBLK
_PROWS = _NBLK * (_TBLK // 4)
_PW = 4 * EMB
_BSH = _TBLK.bit_length() - 1
_QSH = _BSH - 2
_QMASK = (_TBLK // 4) - 1


def _retile_body(eye, utabT, ptabT, uout, pout):
  f32 = jnp.float32
  ua = utabT[...]
  pa = ptabT[...]
  q = _TBLK // 4
  uin = jnp.concatenate([ua[:, k * q:(k + 1) * q] for k in range(4)], axis=0)
  pin = jnp.concatenate([pa[:, k * q:(k + 1) * q] for k in range(4)], axis=0)
  uout[...] = lax.dot_general(
      uin, eye[...], dimension_numbers=(((0,), (0,)), ((), ())),
      preferred_element_type=f32)
  pout[...] = lax.dot_general(
      pin, eye[...], dimension_numbers=(((0,), (0,)), ((), ())),
      preferred_element_type=f32)


@jax.jit
def _retile(utabT, ptabT):
  eye = jnp.eye(_PW, dtype=jnp.float32)
  in_spec = pl.BlockSpec((EMB, _TBLK), lambda i: (0, i))
  out_spec = pl.BlockSpec((_TBLK // 4, _PW), lambda i: (i, 0))
  return pl.pallas_call(
      _retile_body,
      grid=(_NBLK,),
      in_specs=[pl.BlockSpec((_PW, _PW), lambda i: (0, 0)), in_spec, in_spec],
      out_specs=[out_spec, out_spec],
      out_shape=[
          jax.ShapeDtypeStruct((_PROWS, _PW), jnp.float32),
          jax.ShapeDtypeStruct((_PROWS, _PW), jnp.float32),
      ],
  )(eye, utabT, ptabT)


def _sc_gather_body(uid_hbm, pid_hbm, utab_hbm, ptab_hbm, uout_hbm, pout_hbm,
                    uidx_v, pidx_v, urows, prows, usem, psem):
  wid = lax.axis_index("s") * _NC + lax.axis_index("c")
  base = wid * _BPW
  pltpu.sync_copy(uid_hbm.at[pl.ds(base, _BPW)], uidx_v)
  pltpu.sync_copy(pid_hbm.at[pl.ds(base, _BPW)], pidx_v)
  for c in range(_BPW // 16):
    s = pl.ds(c * 16, 16)
    vu = uidx_v[s]
    uidx_v[s] = jnp.bitwise_or(
        lax.shift_left(lax.shift_right_logical(vu, _BSH), _QSH),
        jnp.bitwise_and(vu, _QMASK))
    vp = pidx_v[s]
    pidx_v[s] = jnp.bitwise_or(
        lax.shift_left(lax.shift_right_logical(vp, _BSH), _QSH),
        jnp.bitwise_and(vp, _QMASK))
  ucopy = pltpu.async_copy(utab_hbm.at[uidx_v], urows, usem)
  pcopy = pltpu.async_copy(ptab_hbm.at[pidx_v], prows, psem)
  ucopy.wait()
  pltpu.sync_copy(urows, uout_hbm.at[pl.ds(base, _BPW)])
  pcopy.wait()
  pltpu.sync_copy(prows, pout_hbm.at[pl.ds(base, _BPW)])


@jax.jit
def _sc_gather(uid, pid, utab, ptab):
  mesh = plsc.VectorSubcoreMesh(core_axis_name="c", subcore_axis_name="s")
  return pl.kernel(
      _sc_gather_body,
      mesh=mesh,
      compiler_params=pltpu.CompilerParams(needs_layout_passes=False),
      out_type=[
          jax.ShapeDtypeStruct((BATCH, _PW), jnp.float32),
          jax.ShapeDtypeStruct((BATCH, _PW), jnp.float32),
      ],
      scratch_types=[
          pltpu.VMEM((_BPW,), jnp.int32),
          pltpu.VMEM((_BPW,), jnp.int32),
          pltpu.VMEM((_BPW, _PW), jnp.float32),
          pltpu.VMEM((_BPW, _PW), jnp.float32),
          pltpu.SemaphoreType.DMA,
          pltpu.SemaphoreType.DMA,
      ],
  )(uid, pid, utab, ptab)


def _quad_select(qid, pack):
  q = jnp.bitwise_and(lax.shift_right_logical(qid[:], _QSH), 3)[:, None]
  v01 = jnp.where(q == 0, pack[:, 0:EMB], pack[:, EMB:2 * EMB])
  v23 = jnp.where(q == 2, pack[:, 2 * EMB:3 * EMB], pack[:, 3 * EMB:4 * EMB])
  return jnp.where(q < 2, v01, v23)


def _mlp_body(uid, pid, upack, ppack, ff, wnum, bnum, wsty, bsty, w1, b1, w2,
              b2, w3, b3, w4, b4, out):
  f32 = jnp.float32
  uvec = _quad_select(uid, upack)
  pvec = _quad_select(pid, ppack)
  numeric = jnp.maximum(
      jnp.dot(ff[:, :NUM_NUMERIC], wnum[:], preferred_element_type=f32)
      + bnum[:], 0.0)
  style = jnp.maximum(
      jnp.dot(ff[:, NUM_NUMERIC:], wsty[:], preferred_element_type=f32)
      + bsty[:], 0.0)
  h = (jnp.dot(uvec, w1[0:EMB], preferred_element_type=f32)
       + jnp.dot(pvec, w1[EMB:2 * EMB], preferred_element_type=f32)
       + jnp.dot(numeric, w1[2 * EMB:3 * EMB], preferred_element_type=f32)
       + jnp.dot(style, w1[3 * EMB:4 * EMB], preferred_element_type=f32)
       + b1[:])
  h = jnp.maximum(h, 0.0)
  h = jnp.maximum(jnp.dot(h, w2[:], preferred_element_type=f32) + b2[:], 0.0)
  h = jnp.maximum(jnp.dot(h, w3[:], preferred_element_type=f32) + b3[:], 0.0)
  logit = jnp.sum(h * w4[:], axis=1, keepdims=True) + b4[:]
  out[:] = 1.0 / (1.0 + jnp.exp(-logit))


@jax.jit
def _mlp(uid, pid, upack, ppack, ff, wnum, bnum, wsty, bsty, w1, b1, w2, b2,
         w3, b3, w4, b4):
  nblk = 4
  blk = BATCH // nblk
  row_spec = lambda width: pl.BlockSpec((blk, width), lambda i: (i, 0))
  id_spec = pl.BlockSpec((blk,), lambda i: (i,))
  full = lambda a: pl.BlockSpec(a.shape, lambda i: tuple(0 for _ in a.shape))
  return pl.pallas_call(
      _mlp_body,
      grid=(nblk,),
      in_specs=[
          id_spec,
          id_spec,
          row_spec(_PW),
          row_spec(_PW),
          row_spec(FEAT),
          full(wnum), full(bnum), full(wsty), full(bsty),
          full(w1), full(b1), full(w2), full(b2), full(w3), full(b3),
          full(w4), full(b4),
      ],
      out_specs=pl.BlockSpec((blk, 1), lambda i: (i, 0)),
      out_shape=jax.ShapeDtypeStruct((BATCH, 1), jnp.float32),
  )(uid, pid, upack, ppack, ff, wnum, bnum, wsty, bsty, w1, b1, w2, b2, w3,
    b3, w4, b4)


def kernel(user_id, product_id, full_features, user_table, product_table,
           W_num, b_num, W_style, b_style, W1, b1, W2, b2, W3, b3, W4, b4):
  uid = user_id.astype(jnp.int32)
  pid = product_id.astype(jnp.int32)
  utabT = jnp.swapaxes(user_table, 0, 1)
  ptabT = jnp.swapaxes(product_table, 0, 1)
  utab, ptab = _retile(utabT, ptabT)
  upack, ppack = _sc_gather(uid, pid, utab, ptab)
  return _mlp(uid, pid, upack, ppack, full_features,
              W_num, b_num.reshape(1, EMB), W_style, b_style.reshape(1, EMB),
              W1, b1.reshape(1, 128), W2, b2.reshape(1, 64),
              W3, b3.reshape(1, 32), W4.reshape(1, 32), b4.reshape(1, 1))


# speedup vs baseline: 1.0565x; 1.0164x over previous
import functools

import jax
import jax.numpy as jnp
from jax import lax
from jax.experimental import pallas as pl
from jax.experimental.pallas import tpu as pltpu
from jax.experimental.pallas import tpu_sc as plsc

NUM_NUMERIC = 64
EMB = 64
BATCH = 4096
FEAT = 128
ROWS = 100000

_NC = 2
_NS = 16
_NW = _NC * _NS
_BPW = BATCH // _NW

_TBLK = 8192
_NBLK = (ROWS + _TBLK - 1) // _TBLK
_PROWS = _NBLK * (_TBLK // 4)
_PW = 4 * EMB
_BSH = _TBLK.bit_length() - 1
_QSH = _BSH - 2
_QMASK = (_TBLK // 4) - 1


def _retile_body(eye, utabT, ptabT, uout, pout):
  f32 = jnp.float32
  ua = utabT[...]
  pa = ptabT[...]
  q = _TBLK // 4
  uin = jnp.concatenate([ua[:, k * q:(k + 1) * q] for k in range(4)], axis=0)
  pin = jnp.concatenate([pa[:, k * q:(k + 1) * q] for k in range(4)], axis=0)
  uout[...] = lax.dot_general(
      uin, eye[...], dimension_numbers=(((0,), (0,)), ((), ())),
      preferred_element_type=f32)
  pout[...] = lax.dot_general(
      pin, eye[...], dimension_numbers=(((0,), (0,)), ((), ())),
      preferred_element_type=f32)


@jax.jit
def _retile(utabT, ptabT):
  eye = jnp.eye(_PW, dtype=jnp.float32)
  in_spec = pl.BlockSpec((EMB, _TBLK), lambda i: (0, i))
  out_spec = pl.BlockSpec((_TBLK // 4, _PW), lambda i: (i, 0))
  return pl.pallas_call(
      _retile_body,
      grid=(_NBLK,),
      in_specs=[pl.BlockSpec((_PW, _PW), lambda i: (0, 0)), in_spec, in_spec],
      out_specs=[out_spec, out_spec],
      out_shape=[
          jax.ShapeDtypeStruct((_PROWS, _PW), jnp.float32),
          jax.ShapeDtypeStruct((_PROWS, _PW), jnp.float32),
      ],
  )(eye, utabT, ptabT)


def _sc_gather_body(uid_hbm, pid_hbm, utab_hbm, ptab_hbm, uout_hbm, pout_hbm,
                    uidx_v, pidx_v, urows, prows, usem, psem):
  wid = lax.axis_index("s") * _NC + lax.axis_index("c")
  base = wid * _BPW
  pltpu.sync_copy(uid_hbm.at[pl.ds(base, _BPW)], uidx_v)
  pltpu.sync_copy(pid_hbm.at[pl.ds(base, _BPW)], pidx_v)
  for c in range(_BPW // 16):
    s = pl.ds(c * 16, 16)
    vu = uidx_v[s]
    uidx_v[s] = jnp.bitwise_or(
        lax.shift_left(lax.shift_right_logical(vu, _BSH), _QSH),
        jnp.bitwise_and(vu, _QMASK))
    vp = pidx_v[s]
    pidx_v[s] = jnp.bitwise_or(
        lax.shift_left(lax.shift_right_logical(vp, _BSH), _QSH),
        jnp.bitwise_and(vp, _QMASK))
  ucopy = pltpu.async_copy(utab_hbm.at[uidx_v], urows, usem)
  pcopy = pltpu.async_copy(ptab_hbm.at[pidx_v], prows, psem)
  ucopy.wait()
  pltpu.sync_copy(urows, uout_hbm.at[pl.ds(base, _BPW)])
  pcopy.wait()
  pltpu.sync_copy(prows, pout_hbm.at[pl.ds(base, _BPW)])


@jax.jit
def _sc_gather(uid, pid, utab, ptab):
  mesh = plsc.VectorSubcoreMesh(core_axis_name="c", subcore_axis_name="s")
  return pl.kernel(
      _sc_gather_body,
      mesh=mesh,
      compiler_params=pltpu.CompilerParams(needs_layout_passes=False),
      out_type=[
          jax.ShapeDtypeStruct((BATCH, _PW), jnp.float32),
          jax.ShapeDtypeStruct((BATCH, _PW), jnp.float32),
      ],
      scratch_types=[
          pltpu.VMEM((_BPW,), jnp.int32),
          pltpu.VMEM((_BPW,), jnp.int32),
          pltpu.VMEM((_BPW, _PW), jnp.float32),
          pltpu.VMEM((_BPW, _PW), jnp.float32),
          pltpu.SemaphoreType.DMA,
          pltpu.SemaphoreType.DMA,
      ],
  )(uid, pid, utab, ptab)


def _quad_select(qid, pack):
  q = jnp.bitwise_and(lax.shift_right_logical(qid[:], _QSH), 3)[:, None]
  v01 = jnp.where(q == 0, pack[:, 0:EMB], pack[:, EMB:2 * EMB])
  v23 = jnp.where(q == 2, pack[:, 2 * EMB:3 * EMB], pack[:, 3 * EMB:4 * EMB])
  return jnp.where(q < 2, v01, v23)


def _mlp_body(uid, pid, upack, ppack, ff, wnum, bnum, wsty, bsty, w1, b1, w2,
              b2, w3, b3, w4, b4, out):
  f32 = jnp.float32
  uvec = _quad_select(uid, upack)
  pvec = _quad_select(pid, ppack)
  numeric = jnp.maximum(
      jnp.dot(ff[:, :NUM_NUMERIC], wnum[:], preferred_element_type=f32)
      + bnum[:], 0.0)
  style = jnp.maximum(
      jnp.dot(ff[:, NUM_NUMERIC:], wsty[:], preferred_element_type=f32)
      + bsty[:], 0.0)
  h = (jnp.dot(uvec, w1[0:EMB], preferred_element_type=f32)
       + jnp.dot(pvec, w1[EMB:2 * EMB], preferred_element_type=f32)
       + jnp.dot(numeric, w1[2 * EMB:3 * EMB], preferred_element_type=f32)
       + jnp.dot(style, w1[3 * EMB:4 * EMB], preferred_element_type=f32)
       + b1[:])
  h = jnp.maximum(h, 0.0)
  h = jnp.maximum(jnp.dot(h, w2[:], preferred_element_type=f32) + b2[:], 0.0)
  h = jnp.maximum(jnp.dot(h, w3[:], preferred_element_type=f32) + b3[:], 0.0)
  logit = (jnp.sum(h * w4[:], axis=1) + b4[0, 0]).reshape(1, -1)
  out[:] = 1.0 / (1.0 + jnp.exp(-logit))


@jax.jit
def _mlp(uid, pid, upack, ppack, ff, wnum, bnum, wsty, bsty, w1, b1, w2, b2,
         w3, b3, w4, b4):
  nblk = 4
  blk = BATCH // nblk
  row_spec = lambda width: pl.BlockSpec((blk, width), lambda i: (i, 0))
  id_spec = pl.BlockSpec((blk,), lambda i: (i,))
  full = lambda a: pl.BlockSpec(a.shape, lambda i: tuple(0 for _ in a.shape))
  return pl.pallas_call(
      _mlp_body,
      grid=(nblk,),
      in_specs=[
          id_spec,
          id_spec,
          row_spec(_PW),
          row_spec(_PW),
          row_spec(FEAT),
          full(wnum), full(bnum), full(wsty), full(bsty),
          full(w1), full(b1), full(w2), full(b2), full(w3), full(b3),
          full(w4), full(b4),
      ],
      out_specs=pl.BlockSpec((1, blk), lambda i: (0, i)),
      out_shape=jax.ShapeDtypeStruct((1, BATCH), jnp.float32),
  )(uid, pid, upack, ppack, ff, wnum, bnum, wsty, bsty, w1, b1, w2, b2, w3,
    b3, w4, b4)


def kernel(user_id, product_id, full_features, user_table, product_table,
           W_num, b_num, W_style, b_style, W1, b1, W2, b2, W3, b3, W4, b4):
  uid = user_id.astype(jnp.int32)
  pid = product_id.astype(jnp.int32)
  utabT = jnp.swapaxes(user_table, 0, 1)
  ptabT = jnp.swapaxes(product_table, 0, 1)
  utab, ptab = _retile(utabT, ptabT)
  upack, ppack = _sc_gather(uid, pid, utab, ptab)
  out = _mlp(uid, pid, upack, ppack, full_features,
             W_num, b_num.reshape(1, EMB), W_style, b_style.reshape(1, EMB),
             W1, b1.reshape(1, 128), W2, b2.reshape(1, 64),
             W3, b3.reshape(1, 32), W4.reshape(1, 32), b4.reshape(1, 1))
  return out.reshape(BATCH, 1)
